# baseline (speedup 1.0000x reference)
import functools
import math

import jax
import jax.numpy as jnp
from jax import lax
from jax.experimental import pallas as pl
from jax.experimental.pallas import tpu as pltpu
from jax.experimental.pallas import tpu_sc as plsc

LANES = 16


@functools.cache
def _build(B, T, V, D, PE_LEN):
    NC, NS = 2, 16
    NW = NC * NS
    PPW = T // NW
    P = 16
    BI = 2
    BG = B // BI
    NPC = PPW // P
    NCH = NPC * BG
    VPR = D // LANES
    NSLOT = 4
    RPC = BI * P
    SCALE = math.sqrt(float(D))

    mesh = plsc.VectorSubcoreMesh(core_axis_name="c", subcore_axis_name="s")

    @functools.partial(
        pl.kernel,
        out_type=jax.ShapeDtypeStruct((B, T, D), jnp.float32),
        mesh=mesh,
        scratch_types=[
            pltpu.VMEM((NCH, RPC), jnp.int32),
            [pltpu.VMEM((P, D), jnp.float32) for _ in range(2)],
            [pltpu.VMEM((RPC, D), jnp.float32)
             for _ in range(NSLOT)],
            pltpu.SemaphoreType.DMA,
            pltpu.SemaphoreType.DMA,
            pltpu.SemaphoreType.DMA,
            pltpu.SemaphoreType.DMA,
        ],
    )
    def emb_kernel(x_ref, table_ref, pe_ref, out_ref,
                   idx_v, pe_v, tok_v, isem, psem, gsem, wsem):
        wid = lax.axis_index("s") * NC + lax.axis_index("c")
        pos_base = wid * PPW

        idx_copy = pltpu.async_copy(x_ref.at[wid], idx_v, isem)

        def pe_start(pc, buf):
            pltpu.async_copy(pe_ref.at[pl.ds(pos_base + pc * P, P)],
                             pe_v[buf], psem)

        def pe_wait(buf):
            pltpu.make_async_copy(pe_ref.at[pl.ds(0, P)], pe_v[buf],
                                  psem).wait()

        def g_start(c, slot):
            pltpu.async_copy(table_ref.at[idx_v.at[c]], tok_v[slot], gsem)

        def g_wait(slot):
            pltpu.make_async_copy(pe_ref.at[pl.ds(0, RPC)], tok_v[slot],
                                  gsem).wait()

        def w_start(c, slot):
            pc = c // BG
            b0 = (c % BG) * BI
            for bi in range(BI):
                pltpu.async_copy(tok_v[slot].at[pl.ds(bi * P, P)],
                                 out_ref.at[b0 + bi,
                                            pl.ds(pos_base + pc * P, P)],
                                 wsem)

        def w_wait(slot):
            pltpu.make_async_copy(tok_v[slot],
                                  out_ref.at[0, pl.ds(0, RPC)], wsem).wait()

        idx_copy.wait()
        pe_start(0, 0)
        pe_start(1, 1)
        g_start(0, 0)
        g_start(1, 1)

        def outer(g, _):
            for ci in range(NSLOT):
                c = g * NSLOT + ci
                pbuf = (ci // BG) % 2

                @pl.when(c >= 2)
                def _():
                    w_wait((ci + 2) % NSLOT)

                @pl.when(c + 2 < NCH)
                def _():
                    g_start(c + 2, (ci + 2) % NSLOT)

                if ci % BG == 0:
                    pe_wait(pbuf)
                g_wait(ci)

                pe_buf = pe_v[pbuf]
                buf = tok_v[ci]

                def rows(r, _):
                    for cv in range(VPR):
                        sl = pl.ds(cv * LANES, LANES)
                        pvreg = pe_buf[r, sl]
                        for bi in range(BI):
                            buf[bi * P + r, sl] = buf[bi * P + r, sl] * SCALE + pvreg
                    return 0

                lax.fori_loop(0, P, rows, 0, unroll=1)

                w_start(c, ci)

                if ci % BG == BG - 1:
                    @pl.when(c // BG + 2 < NPC)
                    def _():
                        pe_start(c // BG + 2, pbuf)
            return 0

        lax.fori_loop(0, NCH // NSLOT, outer, 0, unroll=1)

        w_wait(2)
        w_wait(3)

    return emb_kernel


def kernel(x, table, pe):
    B, T = x.shape
    V, D = table.shape
    NW = 32
    P = 16
    BI = 2
    NPC = (T // NW) // P
    x4 = (x.astype(jnp.int32)
           .reshape(B // BI, BI, NW, NPC, P)
           .transpose(2, 3, 0, 1, 4)
           .reshape(NW, NPC * (B // BI), BI * P))
    return _build(B, T, V, D, pe.shape[0])(x4, table, pe)


# speedup vs baseline: 1.7805x; 1.7805x over previous
import functools
import math

import jax
import jax.numpy as jnp
from jax import lax
from jax.experimental import pallas as pl
from jax.experimental.pallas import tpu as pltpu
from jax.experimental.pallas import tpu_sc as plsc

LANES = 16


@functools.cache
def _build(B, T, V, D, PE_LEN):
    NC, NS = 2, 16
    NW = NC * NS
    PPW = T // NW
    P = 8
    NCH = PPW // P
    VPR = D // LANES
    NSLOT = 4
    RPC = B * P
    SCALE = math.sqrt(float(D))

    mesh = plsc.VectorSubcoreMesh(core_axis_name="c", subcore_axis_name="s")

    @functools.partial(
        pl.kernel,
        out_type=jax.ShapeDtypeStruct((B, T, D), jnp.float32),
        mesh=mesh,
        scratch_types=[
            pltpu.VMEM((NCH, RPC), jnp.int32),
            [pltpu.VMEM((P, D), jnp.float32) for _ in range(2)],
            [pltpu.VMEM((RPC, D), jnp.float32)
             for _ in range(NSLOT)],
            pltpu.SemaphoreType.DMA,
            pltpu.SemaphoreType.DMA,
            pltpu.SemaphoreType.DMA,
            pltpu.SemaphoreType.DMA,
        ],
    )
    def emb_kernel(x_ref, table_ref, pe_ref, out_ref,
                   idx_v, pe_v, tok_v, isem, psem, gsem, wsem):
        wid = lax.axis_index("s") * NC + lax.axis_index("c")
        pos_base = wid * PPW

        idx_copy = pltpu.async_copy(x_ref.at[wid], idx_v, isem)

        def pe_start(c, buf):
            pltpu.async_copy(pe_ref.at[pl.ds(pos_base + c * P, P)],
                             pe_v[buf], psem)

        def pe_wait(buf):
            pltpu.make_async_copy(pe_ref.at[pl.ds(0, P)], pe_v[buf],
                                  psem).wait()

        def g_start(c, slot):
            pltpu.async_copy(table_ref.at[idx_v.at[c]], tok_v[slot], gsem)

        def g_wait(slot):
            pltpu.make_async_copy(pe_ref.at[pl.ds(0, RPC)], tok_v[slot],
                                  gsem).wait()

        def w_start(c, slot):
            for b in range(B):
                pltpu.async_copy(tok_v[slot].at[pl.ds(b * P, P)],
                                 out_ref.at[b, pl.ds(pos_base + c * P, P)],
                                 wsem)

        def w_wait(slot):
            pltpu.make_async_copy(tok_v[slot],
                                  out_ref.at[0, pl.ds(0, RPC)], wsem).wait()

        idx_copy.wait()
        pe_start(0, 0)
        pe_start(1, 1)
        g_start(0, 0)
        g_start(1, 1)

        def outer(g, _):
            for ci in range(NSLOT):
                c = g * NSLOT + ci

                @pl.when(c >= 2)
                def _():
                    w_wait((ci + 2) % NSLOT)

                @pl.when(c + 2 < NCH)
                def _():
                    g_start(c + 2, (ci + 2) % NSLOT)

                pe_wait(ci % 2)
                g_wait(ci)

                pe_buf = pe_v[ci % 2]
                buf = tok_v[ci]

                def rows(r, _):
                    for cv in range(VPR):
                        sl = pl.ds(cv * LANES, LANES)
                        pvreg = pe_buf[r, sl]
                        for b in range(B):
                            buf[b * P + r, sl] = buf[b * P + r, sl] * SCALE + pvreg
                    return 0

                lax.fori_loop(0, P, rows, 0, unroll=1)

                w_start(c, ci)

                @pl.when(c + 2 < NCH)
                def _():
                    pe_start(c + 2, ci % 2)
            return 0

        lax.fori_loop(0, NCH // NSLOT, outer, 0, unroll=1)

        w_wait(2)
        w_wait(3)

    return emb_kernel


def kernel(x, table, pe):
    B, T = x.shape
    V, D = table.shape
    NW = 32
    PPW = T // NW
    P = 8
    NCH = PPW // P
    x4 = (x.astype(jnp.int32)
           .reshape(B, NW, NCH, P)
           .transpose(1, 2, 0, 3)
           .reshape(NW, NCH, B * P))
    return _build(B, T, V, D, pe.shape[0])(x4, table, pe)
